# baseline (speedup 1.0000x reference)
import functools

import numpy as np
import jax
import jax.numpy as jnp
from jax import lax
from jax.experimental import pallas as pl
from jax.experimental.pallas import tpu as pltpu

LRELU_SLOPE = 0.01

_STAGES = (
    (6, 1, 2, 1),
    (14, 2, 2, 2),
    (30, 2, 2, 2),
    (62, 2, 1, 1),
)
_PITCH = (8, 16, 32, 64)
_PITCH_OUT = (16, 32, 64, 128)
_QCH = 7


def _lrelu(x):
    return jnp.where(x > 0, x, LRELU_SLOPE * x)


def _fold_gmatrix(w_t, groups, scale, shift, Din, pitch, pitch_out):
    in_ch, outg = w_t.shape[0], w_t.shape[1]
    cing = in_ch // groups
    cout = groups * outg
    Wout = 2 * Din + 2
    dense = jnp.zeros((in_ch, cout, 3, 3, 3), jnp.float32)
    for g in range(groups):
        dense = dense.at[g * cing:(g + 1) * cing, g * outg:(g + 1) * outg].set(
            w_t[g * cing:(g + 1) * cing].astype(jnp.float32))
    dense = dense * scale.astype(jnp.float32)[None, :, None, None, None]
    Sd = np.zeros((2, 2, 3), np.float32)
    for p in range(2):
        for k in range(3):
            e = (p - k) // 2 + 1
            if 0 <= e < 2:
                Sd[p, e, k] = 1.0
    Sw = np.zeros((pitch, pitch_out, 3), np.float32)
    for ow in range(pitch_out):
        for k in range(3):
            j = ow - k
            if 0 <= j < 2 * Din:
                Sw[j // 2, ow, k] = 1.0
    g8 = jnp.einsum("pea,qfb,ioc,xyabc->efxipqyo",
                    jnp.asarray(Sd), jnp.asarray(Sd), jnp.asarray(Sw), dense,
                    precision=lax.Precision.HIGHEST)
    G = g8.reshape(4 * in_ch * pitch, 4 * cout * pitch_out)
    sh8 = jnp.zeros((2, 2, cout, pitch_out), jnp.float32)
    sh8 = sh8.at[:, :, :, :Wout].set(
        shift.astype(jnp.float32)[None, None, :, None])
    return G, sh8.reshape(1, 4 * cout * pitch_out)


def _mlp_body(z_ref, xt_ref, w1z_ref, w1x_ref, b1_ref, w2_ref, b2_ref, o_ref):
    xm = jnp.mean(xt_ref[...], axis=1)
    h = (jnp.dot(z_ref[...], w1z_ref[...], preferred_element_type=jnp.float32)
         + jnp.dot(xm, w1x_ref[...], preferred_element_type=jnp.float32)
         + b1_ref[...])
    h = _lrelu(h)
    o = jnp.dot(h, w2_ref[...], preferred_element_type=jnp.float32) + b2_ref[...]
    o_ref[...] = _lrelu(o)


def _mlp(z, xt, w1z, w1x, b1, w2, b2):
    B = z.shape[0]
    P = w2.shape[1]
    BB = 16
    return pl.pallas_call(
        _mlp_body,
        grid=(B // BB,),
        in_specs=[
            pl.BlockSpec((BB, z.shape[1]), lambda i: (i, 0)),
            pl.BlockSpec((BB, xt.shape[1], xt.shape[2]), lambda i: (i, 0, 0)),
            pl.BlockSpec(w1z.shape, lambda i: (0, 0)),
            pl.BlockSpec(w1x.shape, lambda i: (0, 0)),
            pl.BlockSpec(b1.shape, lambda i: (0, 0)),
            pl.BlockSpec(w2.shape, lambda i: (0, 0)),
            pl.BlockSpec(b2.shape, lambda i: (0, 0)),
        ],
        out_specs=pl.BlockSpec((BB, P), lambda i: (i, 0)),
        out_shape=jax.ShapeDtypeStruct((B, P), jnp.float32),
        compiler_params=pltpu.CompilerParams(
            dimension_semantics=("parallel",)),
    )(z, xt, w1z, w1x, b1, w2, b2)


def _decoder_body(xf1_ref, g1, sh1, g2, sh2, g3, sh3, g4, sh4, o_ref,
                  xf2, xf3, xf4):
    c = pl.program_id(1)

    @pl.when(c == 0)
    def _prologue():
        xf2[...] = jnp.zeros((264, 128), jnp.float32)
        xf3[...] = jnp.zeros((1032, 128), jnp.float32)
        xf4[...] = jnp.zeros((4160, 128), jnp.float32)

        lhs = jnp.concatenate(
            [xf1_ref[pl.ds(s, 56), 0:8] for s in (0, 1, 8, 9)], axis=1)
        r = _lrelu(jnp.dot(lhs, g1[...], preferred_element_type=jnp.float32)
                   + sh1[...])
        for qd in range(7):
            for pd in range(2):
                for ph in range(2):
                    blk = (pd * 2 + ph) * 32
                    xf2[pl.ds((2 * qd + pd + 1) * 16 + ph + 1, 7, 2), 0:32] = \
                        r[qd * 8:qd * 8 + 7, blk:blk + 32]

        lhs = jnp.concatenate(
            [xf2[pl.ds(s, 240), 0:32] for s in (0, 1, 16, 17)], axis=1)
        r = _lrelu(jnp.dot(lhs, g2[...], preferred_element_type=jnp.float32)
                   + sh2[...])
        for qd in range(15):
            for pd in range(2):
                for ph in range(2):
                    blk = (pd * 2 + ph) * 64
                    xf3[pl.ds((2 * qd + pd + 1) * 32 + ph + 1, 15, 2), 0:64] = \
                        r[qd * 16:qd * 16 + 15, blk:blk + 64]

        lhs = jnp.concatenate(
            [xf3[pl.ds(s, 992), 0:64] for s in (0, 1, 32, 33)], axis=1)
        r = _lrelu(jnp.dot(lhs, g3[...], preferred_element_type=jnp.float32)
                   + sh3[...])
        for qd in range(31):
            for pd in range(2):
                for ph in range(2):
                    blk = (pd * 2 + ph) * 128
                    xf4[pl.ds((2 * qd + pd + 1) * 64 + ph + 1, 31, 2), :] = \
                        r[qd * 32:qd * 32 + 31, blk:blk + 128]

    x = xf4[pl.ds(c * (_QCH * 64), 576), :]
    lhs = jnp.concatenate([x[s:s + 448, :] for s in (0, 1, 64, 65)], axis=1)
    r = jax.nn.sigmoid(jnp.dot(lhs, g4[...],
                               preferred_element_type=jnp.float32) + sh4[...])
    for j in range(_QCH):
        for pd in range(2):
            for ph in range(2):
                blk = (pd * 2 + ph) * 128
                o_ref[2 * j + pd, pl.ds(ph, 63, 2), :] = \
                    r[j * 64:j * 64 + 63, blk:blk + 126]


def kernel(w1z, w1x, b1, w2, b2, w_c1, b_c1, w_c2, b_c2, w_c3, b_c3,
           w_c4, b_c4, bn1_scale, bn1_shift, bn2_scale, bn2_shift,
           bn3_scale, bn3_shift, z_sample, x_target):
    B = x_target.shape[0]

    convs = [
        (w_c1, bn1_scale, bn1_shift + b_c1 * bn1_scale),
        (w_c2, bn2_scale, bn2_shift + b_c2 * bn2_scale),
        (w_c3, bn3_scale, bn3_shift + b_c3 * bn3_scale),
        (w_c4, jnp.ones_like(b_c4), b_c4),
    ]
    gs = []
    for (Din, cin, cout, groups), (wt, sc, sh), p, po in zip(
            _STAGES, convs, _PITCH, _PITCH_OUT):
        gs.extend(_fold_gmatrix(wt, groups, sc, sh, Din, p, po))

    h = _mlp(z_sample, x_target, w1z, w1x, b1, w2, b2)
    v = h.reshape(B, 6, 6, 6)
    v = jnp.pad(v, ((0, 0), (1, 1), (1, 1), (0, 2)))
    xf1 = jnp.pad(v.reshape(B, 64, 8), ((0, 0), (0, 8), (0, 120)))

    out = pl.pallas_call(
        _decoder_body,
        grid=(B, 9),
        in_specs=[
            pl.BlockSpec((None, 72, 128), lambda b, c: (b, 0, 0)),
            pl.BlockSpec((32, 128), lambda b, c: (0, 0)),
            pl.BlockSpec((1, 128), lambda b, c: (0, 0)),
            pl.BlockSpec((128, 256), lambda b, c: (0, 0)),
            pl.BlockSpec((1, 256), lambda b, c: (0, 0)),
            pl.BlockSpec((256, 512), lambda b, c: (0, 0)),
            pl.BlockSpec((1, 512), lambda b, c: (0, 0)),
            pl.BlockSpec((512, 512), lambda b, c: (0, 0)),
            pl.BlockSpec((1, 512), lambda b, c: (0, 0)),
        ],
        out_specs=pl.BlockSpec((None, None, 2 * _QCH, 126, 126),
                               lambda b, c: (b, 0, c, 0, 0)),
        out_shape=jax.ShapeDtypeStruct((B, 1, 126, 126, 126), jnp.float32),
        scratch_shapes=[
            pltpu.VMEM((264, 128), jnp.float32),
            pltpu.VMEM((1032, 128), jnp.float32),
            pltpu.VMEM((4160, 128), jnp.float32),
        ],
        compiler_params=pltpu.CompilerParams(
            dimension_semantics=("parallel", "arbitrary")),
    )(xf1, *gs)
    return out


# speedup vs baseline: 2.6108x; 2.6108x over previous
import functools

import numpy as np
import jax
import jax.numpy as jnp
from jax import lax
from jax.experimental import pallas as pl
from jax.experimental.pallas import tpu as pltpu

LRELU_SLOPE = 0.01

_STAGES = (
    (6, 1, 2, 1),
    (14, 2, 2, 2),
    (30, 2, 2, 2),
    (62, 2, 1, 1),
)
_PITCH = (8, 16, 32, 64)
_PITCH_OUT = (16, 32, 64, 128)
_QCH = 7


def _lrelu(x):
    return jnp.where(x > 0, x, LRELU_SLOPE * x)


def _fold_gmatrix(w_t, groups, scale, shift, Din, pitch, pitch_out):
    in_ch, outg = w_t.shape[0], w_t.shape[1]
    cing = in_ch // groups
    cout = groups * outg
    Wout = 2 * Din + 2
    dense = jnp.zeros((in_ch, cout, 3, 3, 3), jnp.float32)
    for g in range(groups):
        dense = dense.at[g * cing:(g + 1) * cing, g * outg:(g + 1) * outg].set(
            w_t[g * cing:(g + 1) * cing].astype(jnp.float32))
    dense = dense * scale.astype(jnp.float32)[None, :, None, None, None]
    Sd = np.zeros((2, 2, 3), np.float32)
    for p in range(2):
        for k in range(3):
            e = (p - k) // 2 + 1
            if 0 <= e < 2:
                Sd[p, e, k] = 1.0
    Sw = np.zeros((pitch, pitch_out, 3), np.float32)
    for ow in range(pitch_out):
        for k in range(3):
            j = ow - k
            if 0 <= j < 2 * Din:
                Sw[j // 2, ow, k] = 1.0
    g8 = jnp.einsum("pea,qfb,ioc,xyabc->efxipqyo",
                    jnp.asarray(Sd), jnp.asarray(Sd), jnp.asarray(Sw), dense,
                    precision=lax.Precision.HIGHEST)
    G = g8.reshape(4 * in_ch * pitch, 4 * cout * pitch_out)
    sh8 = jnp.zeros((2, 2, cout, pitch_out), jnp.float32)
    sh8 = sh8.at[:, :, :, :Wout].set(
        shift.astype(jnp.float32)[None, None, :, None])
    return G, sh8.reshape(1, 4 * cout * pitch_out)


def _mlp_body(z_ref, xt_ref, w1z_ref, w1x_ref, b1_ref, w2_ref, b2_ref, o_ref):
    xm = jnp.mean(xt_ref[...], axis=1)
    h = (jnp.dot(z_ref[...], w1z_ref[...], preferred_element_type=jnp.float32)
         + jnp.dot(xm, w1x_ref[...], preferred_element_type=jnp.float32)
         + b1_ref[...])
    h = _lrelu(h)
    o = jnp.dot(h, w2_ref[...], preferred_element_type=jnp.float32) + b2_ref[...]
    o_ref[...] = _lrelu(o)


def _mlp(z, xt, w1z, w1x, b1, w2, b2):
    B = z.shape[0]
    P = w2.shape[1]
    BB = 16
    return pl.pallas_call(
        _mlp_body,
        grid=(B // BB,),
        in_specs=[
            pl.BlockSpec((BB, z.shape[1]), lambda i: (i, 0)),
            pl.BlockSpec((BB, xt.shape[1], xt.shape[2]), lambda i: (i, 0, 0)),
            pl.BlockSpec(w1z.shape, lambda i: (0, 0)),
            pl.BlockSpec(w1x.shape, lambda i: (0, 0)),
            pl.BlockSpec(b1.shape, lambda i: (0, 0)),
            pl.BlockSpec(w2.shape, lambda i: (0, 0)),
            pl.BlockSpec(b2.shape, lambda i: (0, 0)),
        ],
        out_specs=pl.BlockSpec((BB, P), lambda i: (i, 0)),
        out_shape=jax.ShapeDtypeStruct((B, P), jnp.float32),
        compiler_params=pltpu.CompilerParams(
            dimension_semantics=("parallel",)),
    )(z, xt, w1z, w1x, b1, w2, b2)


def _decoder_body(xf1_ref, g1, sh1, g2, sh2, g3, sh3, g4, sh4, o_ref,
                  xf2, xf3, xf4):
    j = pl.program_id(1)

    @pl.when(j == 0)
    def _prologue():
        xf4[...] = jnp.zeros((8136, 128), jnp.float32)
        for i in range(2):
            base = 4032 * i
            xf2[...] = jnp.zeros((264, 128), jnp.float32)
            xf3[...] = jnp.zeros((1032, 128), jnp.float32)

            lhs = jnp.concatenate(
                [xf1_ref[i, pl.ds(s, 56), 0:8] for s in (0, 1, 8, 9)], axis=1)
            r = _lrelu(jnp.dot(lhs, g1[...],
                               preferred_element_type=jnp.float32) + sh1[...])
            for qd in range(7):
                for pd in range(2):
                    for ph in range(2):
                        blk = (pd * 2 + ph) * 32
                        xf2[pl.ds((2 * qd + pd + 1) * 16 + ph + 1, 7, 2),
                            0:32] = r[qd * 8:qd * 8 + 7, blk:blk + 32]

            lhs = jnp.concatenate(
                [xf2[pl.ds(s, 240), 0:32] for s in (0, 1, 16, 17)], axis=1)
            r = _lrelu(jnp.dot(lhs, g2[...],
                               preferred_element_type=jnp.float32) + sh2[...])
            for qd in range(15):
                for pd in range(2):
                    for ph in range(2):
                        blk = (pd * 2 + ph) * 64
                        xf3[pl.ds((2 * qd + pd + 1) * 32 + ph + 1, 15, 2),
                            0:64] = r[qd * 16:qd * 16 + 15, blk:blk + 64]

            lhs = jnp.concatenate(
                [xf3[pl.ds(s, 992), 0:64] for s in (0, 1, 32, 33)], axis=1)
            r = _lrelu(jnp.dot(lhs, g3[...],
                               preferred_element_type=jnp.float32) + sh3[...])
            for qd in range(31):
                for pd in range(2):
                    for ph in range(2):
                        blk = (pd * 2 + ph) * 128
                        xf4[pl.ds(base + (2 * qd + pd + 1) * 64 + ph + 1,
                                  31, 2), :] = \
                            r[qd * 32:qd * 32 + 31, blk:blk + 128]

    x = xf4[pl.ds(j * 896, 968), :]
    lhs = jnp.concatenate([x[s:s + 896, :] for s in (0, 1, 64, 65)], axis=1)
    r = jax.nn.sigmoid(jnp.dot(lhs, g4[...],
                               preferred_element_type=jnp.float32) + sh4[...])
    for jj in range(14):
        for pd in range(2):
            m = 2 * jj + pd
            for ph in range(2):
                blk = (pd * 2 + ph) * 128
                o_ref[pl.ds(m * 126 + ph, 63, 2), :] = \
                    r[jj * 64:jj * 64 + 63, blk:blk + 126]


def kernel(w1z, w1x, b1, w2, b2, w_c1, b_c1, w_c2, b_c2, w_c3, b_c3,
           w_c4, b_c4, bn1_scale, bn1_shift, bn2_scale, bn2_shift,
           bn3_scale, bn3_shift, z_sample, x_target):
    B = x_target.shape[0]

    convs = [
        (w_c1, bn1_scale, bn1_shift + b_c1 * bn1_scale),
        (w_c2, bn2_scale, bn2_shift + b_c2 * bn2_scale),
        (w_c3, bn3_scale, bn3_shift + b_c3 * bn3_scale),
        (w_c4, jnp.ones_like(b_c4), b_c4),
    ]
    gs = []
    for (Din, cin, cout, groups), (wt, sc, sh), p, po in zip(
            _STAGES, convs, _PITCH, _PITCH_OUT):
        gs.extend(_fold_gmatrix(wt, groups, sc, sh, Din, p, po))

    h = _mlp(z_sample, x_target, w1z, w1x, b1, w2, b2)
    v = h.reshape(B, 6, 6, 6)
    v = jnp.pad(v, ((0, 0), (1, 1), (1, 1), (0, 2)))
    xf1 = jnp.pad(v.reshape(B, 64, 8), ((0, 0), (0, 8), (0, 120)))
    xf1 = xf1.reshape(B // 2, 2, 72, 128)

    out = pl.pallas_call(
        _decoder_body,
        grid=(B // 2, 9),
        in_specs=[
            pl.BlockSpec((None, 2, 72, 128), lambda p, c: (p, 0, 0, 0)),
            pl.BlockSpec((32, 128), lambda p, c: (0, 0)),
            pl.BlockSpec((1, 128), lambda p, c: (0, 0)),
            pl.BlockSpec((128, 256), lambda p, c: (0, 0)),
            pl.BlockSpec((1, 256), lambda p, c: (0, 0)),
            pl.BlockSpec((256, 512), lambda p, c: (0, 0)),
            pl.BlockSpec((1, 512), lambda p, c: (0, 0)),
            pl.BlockSpec((512, 512), lambda p, c: (0, 0)),
            pl.BlockSpec((1, 512), lambda p, c: (0, 0)),
        ],
        out_specs=pl.BlockSpec((28 * 126, 126),
                               lambda p, c: (p * 9 + c, 0)),
        out_shape=jax.ShapeDtypeStruct((B * 126 * 126, 126), jnp.float32),
        scratch_shapes=[
            pltpu.VMEM((264, 128), jnp.float32),
            pltpu.VMEM((1032, 128), jnp.float32),
            pltpu.VMEM((8136, 128), jnp.float32),
        ],
        compiler_params=pltpu.CompilerParams(
            dimension_semantics=("parallel", "arbitrary")),
    )(xf1, *gs)
    return out.reshape(B, 1, 126, 126, 126)


# speedup vs baseline: 2.7298x; 1.0456x over previous
import functools

import numpy as np
import jax
import jax.numpy as jnp
from jax import lax
from jax.experimental import pallas as pl
from jax.experimental.pallas import tpu as pltpu

LRELU_SLOPE = 0.01

_STAGES = (
    (6, 1, 2, 1),
    (14, 2, 2, 2),
    (30, 2, 2, 2),
    (62, 2, 1, 1),
)
_PITCH = (8, 16, 32, 64)
_PITCH_OUT = (16, 32, 64, 128)
_QCH = 7


def _lrelu(x):
    return jnp.where(x > 0, x, LRELU_SLOPE * x)


def _fold_gmatrix(w_t, groups, scale, shift, Din, pitch, pitch_out):
    in_ch, outg = w_t.shape[0], w_t.shape[1]
    cing = in_ch // groups
    cout = groups * outg
    Wout = 2 * Din + 2
    dense = jnp.zeros((in_ch, cout, 3, 3, 3), jnp.float32)
    for g in range(groups):
        dense = dense.at[g * cing:(g + 1) * cing, g * outg:(g + 1) * outg].set(
            w_t[g * cing:(g + 1) * cing].astype(jnp.float32))
    dense = dense * scale.astype(jnp.float32)[None, :, None, None, None]
    Sd = np.zeros((2, 2, 3), np.float32)
    for p in range(2):
        for k in range(3):
            e = (p - k) // 2 + 1
            if 0 <= e < 2:
                Sd[p, e, k] = 1.0
    Sw = np.zeros((pitch, pitch_out, 3), np.float32)
    for ow in range(pitch_out):
        for k in range(3):
            j = ow - k
            if 0 <= j < 2 * Din:
                Sw[j // 2, ow, k] = 1.0
    g8 = jnp.einsum("pea,qfb,ioc,xyabc->efxipqyo",
                    jnp.asarray(Sd), jnp.asarray(Sd), jnp.asarray(Sw), dense,
                    precision=lax.Precision.HIGHEST)
    G = g8.reshape(4 * in_ch * pitch, 4 * cout * pitch_out)
    sh8 = jnp.zeros((2, 2, cout, pitch_out), jnp.float32)
    sh8 = sh8.at[:, :, :, :Wout].set(
        shift.astype(jnp.float32)[None, None, :, None])
    return G, sh8.reshape(1, 4 * cout * pitch_out)


def _mlp_body(z_ref, xt_ref, w1z_ref, w1x_ref, b1_ref, w2_ref, b2_ref, o_ref):
    xm = jnp.mean(xt_ref[...], axis=1)
    h = (jnp.dot(z_ref[...], w1z_ref[...], preferred_element_type=jnp.float32)
         + jnp.dot(xm, w1x_ref[...], preferred_element_type=jnp.float32)
         + b1_ref[...])
    h = _lrelu(h)
    o = jnp.dot(h, w2_ref[...], preferred_element_type=jnp.float32) + b2_ref[...]
    o_ref[...] = _lrelu(o)


def _mlp(z, xt, w1z, w1x, b1, w2, b2):
    B = z.shape[0]
    P = w2.shape[1]
    BB = 16
    return pl.pallas_call(
        _mlp_body,
        grid=(B // BB,),
        in_specs=[
            pl.BlockSpec((BB, z.shape[1]), lambda i: (i, 0)),
            pl.BlockSpec((BB, xt.shape[1], xt.shape[2]), lambda i: (i, 0, 0)),
            pl.BlockSpec(w1z.shape, lambda i: (0, 0)),
            pl.BlockSpec(w1x.shape, lambda i: (0, 0)),
            pl.BlockSpec(b1.shape, lambda i: (0, 0)),
            pl.BlockSpec(w2.shape, lambda i: (0, 0)),
            pl.BlockSpec(b2.shape, lambda i: (0, 0)),
        ],
        out_specs=pl.BlockSpec((BB, P), lambda i: (i, 0)),
        out_shape=jax.ShapeDtypeStruct((B, P), jnp.float32),
        compiler_params=pltpu.CompilerParams(
            dimension_semantics=("parallel",)),
    )(z, xt, w1z, w1x, b1, w2, b2)


def _decoder_body(xf1_ref, g1, sh1, g2, sh2, g3, sh3, g4, sh4, o_ref,
                  xf2, xf3, xf4):
    j = pl.program_id(1)

    @pl.when(j == 0)
    def _prologue():
        xf4[...] = jnp.zeros((8136, 128), jnp.float32)
        for i in range(2):
            base = 4032 * i
            xf2[...] = jnp.zeros((264, 128), jnp.float32)
            xf3[...] = jnp.zeros((1032, 128), jnp.float32)

            lhs = jnp.concatenate(
                [xf1_ref[i, pl.ds(s, 56), 0:8] for s in (0, 1, 8, 9)], axis=1)
            r = _lrelu(jnp.dot(lhs, g1[...],
                               preferred_element_type=jnp.float32) + sh1[...])
            for qd in range(7):
                for pd in range(2):
                    for ph in range(2):
                        blk = (pd * 2 + ph) * 32
                        xf2[pl.ds((2 * qd + pd + 1) * 16 + ph + 1, 7, 2),
                            0:32] = r[qd * 8:qd * 8 + 7, blk:blk + 32]

            lhs = jnp.concatenate(
                [xf2[pl.ds(s, 240), 0:32] for s in (0, 1, 16, 17)], axis=1)
            r = _lrelu(jnp.dot(lhs, g2[...],
                               preferred_element_type=jnp.float32) + sh2[...])
            for qd in range(15):
                for pd in range(2):
                    for ph in range(2):
                        blk = (pd * 2 + ph) * 64
                        xf3[pl.ds((2 * qd + pd + 1) * 32 + ph + 1, 15, 2),
                            0:64] = r[qd * 16:qd * 16 + 15, blk:blk + 64]

            lhs = jnp.concatenate(
                [xf3[pl.ds(s, 992), 0:64] for s in (0, 1, 32, 33)], axis=1)
            r = _lrelu(jnp.dot(lhs, g3[...],
                               preferred_element_type=jnp.float32) + sh3[...])
            for qd in range(31):
                for pd in range(2):
                    for ph in range(2):
                        blk = (pd * 2 + ph) * 128
                        xf4[pl.ds(base + (2 * qd + pd + 1) * 64 + ph + 1,
                                  31, 2), :] = \
                            r[qd * 32:qd * 32 + 31, blk:blk + 128]

    x = xf4[pl.ds(j * 896, 968), :]
    lhs = jnp.concatenate([x[s:s + 896, :] for s in (0, 1, 64, 65)], axis=1)
    acc = jnp.dot(lhs, g4[...], preferred_element_type=jnp.float32) + sh4[...]
    r = 0.5 + 0.5 * jnp.tanh(0.5 * acc)
    for jj in range(14):
        for pd in range(2):
            m = 2 * jj + pd
            for ph in range(2):
                blk = (pd * 2 + ph) * 128
                o_ref[pl.ds(m * 126 + ph, 63, 2), :] = \
                    r[jj * 64:jj * 64 + 63, blk:blk + 126]


def kernel(w1z, w1x, b1, w2, b2, w_c1, b_c1, w_c2, b_c2, w_c3, b_c3,
           w_c4, b_c4, bn1_scale, bn1_shift, bn2_scale, bn2_shift,
           bn3_scale, bn3_shift, z_sample, x_target):
    B = x_target.shape[0]

    convs = [
        (w_c1, bn1_scale, bn1_shift + b_c1 * bn1_scale),
        (w_c2, bn2_scale, bn2_shift + b_c2 * bn2_scale),
        (w_c3, bn3_scale, bn3_shift + b_c3 * bn3_scale),
        (w_c4, jnp.ones_like(b_c4), b_c4),
    ]
    gs = []
    for (Din, cin, cout, groups), (wt, sc, sh), p, po in zip(
            _STAGES, convs, _PITCH, _PITCH_OUT):
        gs.extend(_fold_gmatrix(wt, groups, sc, sh, Din, p, po))

    h = _mlp(z_sample, x_target, w1z, w1x, b1, w2, b2)
    v = h.reshape(B, 6, 6, 6)
    v = jnp.pad(v, ((0, 0), (1, 1), (1, 1), (0, 2)))
    xf1 = jnp.pad(v.reshape(B, 64, 8), ((0, 0), (0, 8), (0, 120)))
    xf1 = xf1.reshape(B // 2, 2, 72, 128)

    out = pl.pallas_call(
        _decoder_body,
        grid=(B // 2, 9),
        in_specs=[
            pl.BlockSpec((None, 2, 72, 128), lambda p, c: (p, 0, 0, 0)),
            pl.BlockSpec((32, 128), lambda p, c: (0, 0)),
            pl.BlockSpec((1, 128), lambda p, c: (0, 0)),
            pl.BlockSpec((128, 256), lambda p, c: (0, 0)),
            pl.BlockSpec((1, 256), lambda p, c: (0, 0)),
            pl.BlockSpec((256, 512), lambda p, c: (0, 0)),
            pl.BlockSpec((1, 512), lambda p, c: (0, 0)),
            pl.BlockSpec((512, 512), lambda p, c: (0, 0)),
            pl.BlockSpec((1, 512), lambda p, c: (0, 0)),
        ],
        out_specs=pl.BlockSpec((28 * 126, 126),
                               lambda p, c: (p * 9 + c, 0)),
        out_shape=jax.ShapeDtypeStruct((B * 126 * 126, 126), jnp.float32),
        scratch_shapes=[
            pltpu.VMEM((264, 128), jnp.float32),
            pltpu.VMEM((1032, 128), jnp.float32),
            pltpu.VMEM((8136, 128), jnp.float32),
        ],
        compiler_params=pltpu.CompilerParams(
            dimension_semantics=("parallel", "arbitrary")),
    )(xf1, *gs)
    return out.reshape(B, 1, 126, 126, 126)


# speedup vs baseline: 3.3381x; 1.2229x over previous
import functools

import numpy as np
import jax
import jax.numpy as jnp
from jax import lax
from jax.experimental import pallas as pl
from jax.experimental.pallas import tpu as pltpu

LRELU_SLOPE = 0.01

_STAGES = (
    (6, 1, 2, 1),
    (14, 2, 2, 2),
    (30, 2, 2, 2),
    (62, 2, 1, 1),
)
_PITCH = (8, 16, 32, 64)
_PITCH_OUT = (16, 32, 64, 128)
_QCH = 7


def _lrelu(x):
    return jnp.where(x > 0, x, LRELU_SLOPE * x)


def _fold_gmatrix(w_t, groups, scale, shift, Din, pitch, pitch_out):
    in_ch, outg = w_t.shape[0], w_t.shape[1]
    cing = in_ch // groups
    cout = groups * outg
    Wout = 2 * Din + 2
    dense = jnp.zeros((in_ch, cout, 3, 3, 3), jnp.float32)
    for g in range(groups):
        dense = dense.at[g * cing:(g + 1) * cing, g * outg:(g + 1) * outg].set(
            w_t[g * cing:(g + 1) * cing].astype(jnp.float32))
    dense = dense * scale.astype(jnp.float32)[None, :, None, None, None]
    Sd = np.zeros((2, 2, 3), np.float32)
    for p in range(2):
        for k in range(3):
            e = (p - k) // 2 + 1
            if 0 <= e < 2:
                Sd[p, e, k] = 1.0
    Sw = np.zeros((pitch, pitch_out, 3), np.float32)
    for ow in range(pitch_out):
        for k in range(3):
            j = ow - k
            if 0 <= j < 2 * Din:
                Sw[j // 2, ow, k] = 1.0
    g8 = jnp.einsum("pea,qfb,ioc,xyabc->efxipqyo",
                    jnp.asarray(Sd), jnp.asarray(Sd), jnp.asarray(Sw), dense,
                    precision=lax.Precision.HIGHEST)
    G = g8.reshape(4 * in_ch * pitch, 4 * cout * pitch_out)
    sh8 = jnp.zeros((2, 2, cout, pitch_out), jnp.float32)
    sh8 = sh8.at[:, :, :, :Wout].set(
        shift.astype(jnp.float32)[None, None, :, None])
    return G, sh8.reshape(1, 4 * cout * pitch_out)


def _mlp_body(z_ref, xt_ref, w1z_ref, w1x_ref, b1_ref, w2_ref, b2_ref, o_ref):
    xm = jnp.mean(xt_ref[...], axis=1)
    h = (jnp.dot(z_ref[...], w1z_ref[...], preferred_element_type=jnp.float32)
         + jnp.dot(xm, w1x_ref[...], preferred_element_type=jnp.float32)
         + b1_ref[...])
    h = _lrelu(h)
    o = jnp.dot(h, w2_ref[...], preferred_element_type=jnp.float32) + b2_ref[...]
    o_ref[...] = _lrelu(o)


def _mlp(z, xt, w1z, w1x, b1, w2, b2):
    B = z.shape[0]
    P = w2.shape[1]
    BB = 16
    return pl.pallas_call(
        _mlp_body,
        grid=(B // BB,),
        in_specs=[
            pl.BlockSpec((BB, z.shape[1]), lambda i: (i, 0)),
            pl.BlockSpec((BB, xt.shape[1], xt.shape[2]), lambda i: (i, 0, 0)),
            pl.BlockSpec(w1z.shape, lambda i: (0, 0)),
            pl.BlockSpec(w1x.shape, lambda i: (0, 0)),
            pl.BlockSpec(b1.shape, lambda i: (0, 0)),
            pl.BlockSpec(w2.shape, lambda i: (0, 0)),
            pl.BlockSpec(b2.shape, lambda i: (0, 0)),
        ],
        out_specs=pl.BlockSpec((BB, P), lambda i: (i, 0)),
        out_shape=jax.ShapeDtypeStruct((B, P), jnp.float32),
        compiler_params=pltpu.CompilerParams(
            dimension_semantics=("parallel",)),
    )(z, xt, w1z, w1x, b1, w2, b2)


def _decoder_body(xf1_ref, g1, sh1, g2, sh2, g3, sh3, g4, sh4, o_ref,
                  xf2, xf3, xf4):
    j = pl.program_id(1)

    @pl.when(j == 0)
    def _prologue():
        xf4[...] = jnp.zeros((8136, 128), jnp.float32)
        for i in range(2):
            base = 4032 * i
            xf2[...] = jnp.zeros((264, 128), jnp.float32)
            xf3[...] = jnp.zeros((1032, 128), jnp.float32)

            lhs = jnp.concatenate(
                [xf1_ref[i, pl.ds(s, 56), 0:8] for s in (0, 1, 8, 9)], axis=1)
            r = _lrelu(jnp.dot(lhs, g1[...],
                               preferred_element_type=jnp.float32) + sh1[...])
            for qd in range(7):
                for pd in range(2):
                    for ph in range(2):
                        blk = (pd * 2 + ph) * 32
                        xf2[pl.ds((2 * qd + pd + 1) * 16 + ph + 1, 7, 2),
                            0:32] = r[qd * 8:qd * 8 + 7, blk:blk + 32]

            lhs = jnp.concatenate(
                [xf2[pl.ds(s, 240), 0:32] for s in (0, 1, 16, 17)], axis=1)
            r = _lrelu(jnp.dot(lhs, g2[...],
                               preferred_element_type=jnp.float32) + sh2[...])
            for qd in range(15):
                for pd in range(2):
                    for ph in range(2):
                        blk = (pd * 2 + ph) * 64
                        xf3[pl.ds((2 * qd + pd + 1) * 32 + ph + 1, 15, 2),
                            0:64] = r[qd * 16:qd * 16 + 15, blk:blk + 64]

            lhs = jnp.concatenate(
                [xf3[pl.ds(s, 992), 0:64] for s in (0, 1, 32, 33)], axis=1)
            r = _lrelu(jnp.dot(lhs, g3[...],
                               preferred_element_type=jnp.float32) + sh3[...])
            for qd in range(31):
                for pd in range(2):
                    for ph in range(2):
                        blk = (pd * 2 + ph) * 128
                        xf4[pl.ds(base + (2 * qd + pd + 1) * 64 + ph + 1,
                                  31, 2), :] = \
                            r[qd * 32:qd * 32 + 31, blk:blk + 128]

    x = xf4[pl.ds(j * 2688, 2760), :]
    lhs = jnp.concatenate([x[s:s + 2688, :] for s in (0, 1, 64, 65)], axis=1)
    acc = jnp.dot(lhs, g4[...], preferred_element_type=jnp.float32) + sh4[...]
    r = 0.5 + 0.5 * jnp.tanh(0.5 * acc)
    for jj in range(42):
        for pd in range(2):
            m = 2 * jj + pd
            for ph in range(2):
                blk = (pd * 2 + ph) * 128
                o_ref[pl.ds(m * 126 + ph, 63, 2), :] = \
                    r[jj * 64:jj * 64 + 63, blk:blk + 126]


def kernel(w1z, w1x, b1, w2, b2, w_c1, b_c1, w_c2, b_c2, w_c3, b_c3,
           w_c4, b_c4, bn1_scale, bn1_shift, bn2_scale, bn2_shift,
           bn3_scale, bn3_shift, z_sample, x_target):
    B = x_target.shape[0]

    convs = [
        (w_c1, bn1_scale, bn1_shift + b_c1 * bn1_scale),
        (w_c2, bn2_scale, bn2_shift + b_c2 * bn2_scale),
        (w_c3, bn3_scale, bn3_shift + b_c3 * bn3_scale),
        (w_c4, jnp.ones_like(b_c4), b_c4),
    ]
    gs = []
    for (Din, cin, cout, groups), (wt, sc, sh), p, po in zip(
            _STAGES, convs, _PITCH, _PITCH_OUT):
        gs.extend(_fold_gmatrix(wt, groups, sc, sh, Din, p, po))

    h = _mlp(z_sample, x_target, w1z, w1x, b1, w2, b2)
    v = h.reshape(B, 6, 6, 6)
    v = jnp.pad(v, ((0, 0), (1, 1), (1, 1), (0, 2)))
    xf1 = jnp.pad(v.reshape(B, 64, 8), ((0, 0), (0, 8), (0, 120)))
    xf1 = xf1.reshape(B // 2, 2, 72, 128)

    out = pl.pallas_call(
        _decoder_body,
        grid=(B // 2, 3),
        in_specs=[
            pl.BlockSpec((None, 2, 72, 128), lambda p, c: (p, 0, 0, 0)),
            pl.BlockSpec((32, 128), lambda p, c: (0, 0)),
            pl.BlockSpec((1, 128), lambda p, c: (0, 0)),
            pl.BlockSpec((128, 256), lambda p, c: (0, 0)),
            pl.BlockSpec((1, 256), lambda p, c: (0, 0)),
            pl.BlockSpec((256, 512), lambda p, c: (0, 0)),
            pl.BlockSpec((1, 512), lambda p, c: (0, 0)),
            pl.BlockSpec((512, 512), lambda p, c: (0, 0)),
            pl.BlockSpec((1, 512), lambda p, c: (0, 0)),
        ],
        out_specs=pl.BlockSpec((84 * 126, 126),
                               lambda p, c: (p * 3 + c, 0)),
        out_shape=jax.ShapeDtypeStruct((B * 126 * 126, 126), jnp.float32),
        scratch_shapes=[
            pltpu.VMEM((264, 128), jnp.float32),
            pltpu.VMEM((1032, 128), jnp.float32),
            pltpu.VMEM((8136, 128), jnp.float32),
        ],
        compiler_params=pltpu.CompilerParams(
            dimension_semantics=("parallel", "arbitrary")),
    )(xf1, *gs)
    return out.reshape(B, 1, 126, 126, 126)


# speedup vs baseline: 3.3478x; 1.0029x over previous
import numpy as np
import jax
import jax.numpy as jnp
from jax import lax
from jax.experimental import pallas as pl
from jax.experimental.pallas import tpu as pltpu

LRELU_SLOPE = 0.01

_STAGES = (
    (6, 1, 2, 1),
    (14, 2, 2, 2),
    (30, 2, 2, 2),
    (62, 2, 1, 1),
)
_PITCH = (8, 16, 32, 64)
_PITCH_OUT = (16, 32, 64, 128)


def _lrelu(x):
    return jnp.where(x > 0, x, LRELU_SLOPE * x)


def _fold_gmatrix(w_t, groups, scale, shift, Din, pitch, pitch_out):
    in_ch, outg = w_t.shape[0], w_t.shape[1]
    cing = in_ch // groups
    cout = groups * outg
    Wout = 2 * Din + 2
    dense = jnp.zeros((in_ch, cout, 3, 3, 3), jnp.float32)
    for g in range(groups):
        dense = dense.at[g * cing:(g + 1) * cing, g * outg:(g + 1) * outg].set(
            w_t[g * cing:(g + 1) * cing].astype(jnp.float32))
    dense = dense * scale.astype(jnp.float32)[None, :, None, None, None]
    Sd = np.zeros((2, 2, 3), np.float32)
    for p in range(2):
        for k in range(3):
            e = (p - k) // 2 + 1
            if 0 <= e < 2:
                Sd[p, e, k] = 1.0
    Sw = np.zeros((pitch, pitch_out, 3), np.float32)
    for ow in range(pitch_out):
        for k in range(3):
            j = ow - k
            if 0 <= j < 2 * Din:
                Sw[j // 2, ow, k] = 1.0
    g8 = jnp.einsum("pea,qfb,ioc,xyabc->efxipqyo",
                    jnp.asarray(Sd), jnp.asarray(Sd), jnp.asarray(Sw), dense,
                    precision=lax.Precision.HIGHEST)
    G = g8.reshape(4 * in_ch * pitch, 4 * cout * pitch_out)
    sh8 = jnp.zeros((2, 2, cout, pitch_out), jnp.float32)
    sh8 = sh8.at[:, :, :, :Wout].set(
        shift.astype(jnp.float32)[None, None, :, None])
    return G, sh8.reshape(1, 4 * cout * pitch_out)


def _mlp_body(z_ref, xt_ref, w1z_ref, w1x_ref, b1_ref, w2_ref, b2_ref, o_ref):
    xm = jnp.mean(xt_ref[...], axis=1)
    h = (jnp.dot(z_ref[...], w1z_ref[...], preferred_element_type=jnp.float32)
         + jnp.dot(xm, w1x_ref[...], preferred_element_type=jnp.float32)
         + b1_ref[...])
    h = _lrelu(h)
    o = jnp.dot(h, w2_ref[...], preferred_element_type=jnp.float32) + b2_ref[...]
    o_ref[...] = _lrelu(o)


def _mlp(z, xt, w1z, w1x, b1, w2, b2):
    B = z.shape[0]
    P = w2.shape[1]
    BB = 16
    return pl.pallas_call(
        _mlp_body,
        grid=(B // BB,),
        in_specs=[
            pl.BlockSpec((BB, z.shape[1]), lambda i: (i, 0)),
            pl.BlockSpec((BB, xt.shape[1], xt.shape[2]), lambda i: (i, 0, 0)),
            pl.BlockSpec(w1z.shape, lambda i: (0, 0)),
            pl.BlockSpec(w1x.shape, lambda i: (0, 0)),
            pl.BlockSpec(b1.shape, lambda i: (0, 0)),
            pl.BlockSpec(w2.shape, lambda i: (0, 0)),
            pl.BlockSpec(b2.shape, lambda i: (0, 0)),
        ],
        out_specs=pl.BlockSpec((BB, P), lambda i: (i, 0)),
        out_shape=jax.ShapeDtypeStruct((B, P), jnp.float32),
        compiler_params=pltpu.CompilerParams(
            dimension_semantics=("parallel",)),
    )(z, xt, w1z, w1x, b1, w2, b2)


def _decoder_body(xf1_ref, g1, sh1, g2, sh2, g3, sh3, g4, sh4, o_ref,
                  xf2, xf3, xf4):
    j = pl.program_id(1)

    @pl.when(j == 0)
    def _prologue():
        xf4[...] = jnp.zeros((8136, 128), jnp.float32)
        for i in range(2):
            base = 4032 * i
            xf2[...] = jnp.zeros((264, 128), jnp.float32)
            xf3[...] = jnp.zeros((1032, 128), jnp.float32)

            lhs = jnp.concatenate(
                [xf1_ref[i, pl.ds(s, 56), 0:8] for s in (0, 1, 8, 9)], axis=1)
            r = _lrelu(jnp.dot(lhs, g1[...],
                               preferred_element_type=jnp.float32) + sh1[...])
            for qd in range(7):
                for pd in range(2):
                    for ph in range(2):
                        blk = (pd * 2 + ph) * 32
                        xf2[pl.ds((2 * qd + pd + 1) * 16 + ph + 1, 7, 2),
                            0:32] = r[qd * 8:qd * 8 + 7, blk:blk + 32]

            lhs = jnp.concatenate(
                [xf2[pl.ds(s, 240), 0:32] for s in (0, 1, 16, 17)], axis=1)
            r = _lrelu(jnp.dot(lhs, g2[...],
                               preferred_element_type=jnp.float32) + sh2[...])
            for qd in range(15):
                for pd in range(2):
                    for ph in range(2):
                        blk = (pd * 2 + ph) * 64
                        xf3[pl.ds((2 * qd + pd + 1) * 32 + ph + 1, 15, 2),
                            0:64] = r[qd * 16:qd * 16 + 15, blk:blk + 64]

            lhs = jnp.concatenate(
                [xf3[pl.ds(s, 992), 0:64] for s in (0, 1, 32, 33)], axis=1)
            r = _lrelu(jnp.dot(lhs, g3[...],
                               preferred_element_type=jnp.float32) + sh3[...])
            for qd in range(31):
                for pd in range(2):
                    for ph in range(2):
                        blk = (pd * 2 + ph) * 128
                        xf4[pl.ds(base + (2 * qd + pd + 1) * 64 + ph + 1,
                                  31, 2), :] = \
                            r[qd * 32:qd * 32 + 31, blk:blk + 128]

    x = xf4[pl.ds(j * 2688, 2760), :]
    lhs = jnp.concatenate([x[s:s + 2688, :] for s in (0, 1, 64, 65)], axis=1)
    acc = jnp.dot(lhs, g4[...], preferred_element_type=jnp.float32) + sh4[...]
    r = 0.5 + 0.5 * jnp.tanh(0.5 * acc)
    for jj in range(42):
        for pd in range(2):
            m = 2 * jj + pd
            for ph in range(2):
                blk = (pd * 2 + ph) * 128
                o_ref[pl.ds(m * 126 + ph, 63, 2), :] = \
                    r[jj * 64:jj * 64 + 63, blk:blk + 126]


def kernel(w1z, w1x, b1, w2, b2, w_c1, b_c1, w_c2, b_c2, w_c3, b_c3,
           w_c4, b_c4, bn1_scale, bn1_shift, bn2_scale, bn2_shift,
           bn3_scale, bn3_shift, z_sample, x_target):
    B = x_target.shape[0]

    convs = [
        (w_c1, bn1_scale, bn1_shift + b_c1 * bn1_scale),
        (w_c2, bn2_scale, bn2_shift + b_c2 * bn2_scale),
        (w_c3, bn3_scale, bn3_shift + b_c3 * bn3_scale),
        (w_c4, jnp.ones_like(b_c4), b_c4),
    ]
    gs = []
    for (Din, cin, cout, groups), (wt, sc, sh), p, po in zip(
            _STAGES, convs, _PITCH, _PITCH_OUT):
        gs.extend(_fold_gmatrix(wt, groups, sc, sh, Din, p, po))

    h = _mlp(z_sample, x_target, w1z, w1x, b1, w2, b2)
    v = h.reshape(B, 6, 6, 6)
    v = jnp.pad(v, ((0, 0), (1, 1), (1, 1), (0, 2)))
    xf1 = jnp.pad(v.reshape(B, 64, 8), ((0, 0), (0, 8), (0, 120)))
    xf1 = xf1.reshape(B // 2, 2, 72, 128)

    out = pl.pallas_call(
        _decoder_body,
        grid=(B // 2, 3),
        in_specs=[
            pl.BlockSpec((None, 2, 72, 128), lambda p, c: (p, 0, 0, 0)),
            pl.BlockSpec((32, 128), lambda p, c: (0, 0)),
            pl.BlockSpec((1, 128), lambda p, c: (0, 0)),
            pl.BlockSpec((128, 256), lambda p, c: (0, 0)),
            pl.BlockSpec((1, 256), lambda p, c: (0, 0)),
            pl.BlockSpec((256, 512), lambda p, c: (0, 0)),
            pl.BlockSpec((1, 512), lambda p, c: (0, 0)),
            pl.BlockSpec((512, 512), lambda p, c: (0, 0)),
            pl.BlockSpec((1, 512), lambda p, c: (0, 0)),
        ],
        out_specs=pl.BlockSpec((84 * 126, 126),
                               lambda p, c: (p * 3 + c, 0)),
        out_shape=jax.ShapeDtypeStruct((B * 126 * 126, 126), jnp.float32),
        scratch_shapes=[
            pltpu.VMEM((264, 128), jnp.float32),
            pltpu.VMEM((1032, 128), jnp.float32),
            pltpu.VMEM((8136, 128), jnp.float32),
        ],
        compiler_params=pltpu.CompilerParams(
            dimension_semantics=("parallel", "arbitrary")),
    )(xf1, *gs)
    return out.reshape(B, 1, 126, 126, 126)
